# baseline (speedup 1.0000x reference)
import functools

import jax
import jax.numpy as jnp
from jax import lax
from jax.experimental import pallas as pl
from jax.experimental.pallas import tpu as pltpu
from jax.experimental.pallas import tpu_sc as plsc

_BATCH = 1024
_M = 64
_P = 3
_DIM = 3
_OUT_DIM = 1024

_NC = 2
_NS = 16
_L = 16
_NW = _NC * _NS
_B_PER_W = _BATCH // _NW
_SLAB = 128

_N_CHUNK = _OUT_DIM // _L
_N_G = 4
_B_PER_G = _B_PER_W // _N_G

_ROWS = _M * (_DIM + 1)
_GBUF = _DIM * _B_PER_G * _OUT_DIM


def _sc_body(ctrl_hbm, uspan_hbm, nut_hbm, out_hbm, ctrl_v, ctrl_f, uspan_v,
             nut_v, idx_v, out_v0, out_v1, sem):
    wid = lax.axis_index("s") * _NC + lax.axis_index("c")
    base = wid * _B_PER_W
    bofs = (wid % (_SLAB // _B_PER_W)) * _B_PER_W

    pltpu.sync_copy(ctrl_hbm.at[:, pl.ds((base // _SLAB) * _SLAB, _SLAB)],
                    ctrl_v)
    pltpu.sync_copy(uspan_hbm, uspan_v)
    pltpu.sync_copy(nut_hbm, nut_v)

    @plsc.parallel_loop(0, _M)
    def _flatten(m):
        for d in range(_DIM + 1):
            for g in range(_B_PER_W // _L):
                ctrl_f[pl.ds(m * 128 + d * 32 + g * _L, _L)] = (
                    ctrl_v[m * (_DIM + 1) + d, pl.ds(bofs + g * _L, _L)])

    @plsc.parallel_loop(0, _N_CHUNK)
    def _pre(c):
        u = uspan_v[pl.ds(c * _L, _L)]
        for j in range(_P + 1):
            idx_v[pl.ds((c * (_P + 1) + j) * _L, _L)] = (u + (j - _P)) * 128

    lane = lax.iota(jnp.int32, _L)

    def run_group(g, buf):
        @plsc.parallel_loop(0, _N_CHUNK)
        def c_body(c):
            mj = [idx_v[pl.ds((c * (_P + 1) + j) * _L, _L)]
                  for j in range(_P + 1)]
            nu = [nut_v[j, pl.ds(c * _L, _L)] for j in range(_P + 1)]
            col = lane + c * _L

            @plsc.parallel_loop(0, _B_PER_G, unroll=2)
            def _bloop(b):
                bv = jnp.full((_L,), g * _B_PER_G + b, jnp.int32)
                mb = [mj[j] + bv for j in range(_P + 1)]
                w = plsc.load_gather(ctrl_f, [mb[0] + _DIM * 32])
                a = nu[0] * w
                denom = a
                num0 = a * plsc.load_gather(ctrl_f, [mb[0]])
                num1 = a * plsc.load_gather(ctrl_f, [mb[0] + 32])
                num2 = a * plsc.load_gather(ctrl_f, [mb[0] + 64])
                for j in range(1, _P + 1):
                    w = plsc.load_gather(ctrl_f, [mb[j] + _DIM * 32])
                    a = nu[j] * w
                    denom = denom + a
                    num0 = num0 + a * plsc.load_gather(ctrl_f, [mb[j]])
                    num1 = num1 + a * plsc.load_gather(ctrl_f, [mb[j] + 32])
                    num2 = num2 + a * plsc.load_gather(ctrl_f, [mb[j] + 64])
                inv = 1.0 / denom
                addr = col + b * _OUT_DIM
                plsc.store_scatter(buf, [addr], num0 * inv)
                plsc.store_scatter(buf, [addr + _B_PER_G * _OUT_DIM],
                                   num1 * inv)
                plsc.store_scatter(buf, [addr + 2 * _B_PER_G * _OUT_DIM],
                                   num2 * inv)

        return [pltpu.async_copy(
            buf.at[pl.ds(d * _B_PER_G * _OUT_DIM, _B_PER_G * _OUT_DIM)],
            out_hbm.at[pl.ds(d * _BATCH * _OUT_DIM
                             + (base + g * _B_PER_G) * _OUT_DIM,
                             _B_PER_G * _OUT_DIM)], sem)
            for d in range(_DIM)]

    bufs = (out_v0, out_v1)
    pending = []
    for g in range(_N_G):
        if g >= 2:
            for cp in pending[g - 2]:
                cp.wait()
        pending.append(run_group(g, bufs[g % 2]))
    for cps in pending[-2:]:
        for cp in cps:
            cp.wait()


@jax.jit
def _sc_eval(ctrl_t, uspan, nut):
    mesh = plsc.VectorSubcoreMesh(core_axis_name="c", subcore_axis_name="s",
                                  num_cores=_NC, num_subcores=_NS)
    f = pl.kernel(
        _sc_body,
        out_type=jax.ShapeDtypeStruct((_DIM * _BATCH * _OUT_DIM,),
                                      jnp.float32),
        mesh=mesh,
        scratch_types=[
            pltpu.VMEM((_ROWS, _SLAB), jnp.float32),
            pltpu.VMEM((_ROWS * _B_PER_W,), jnp.float32),
            pltpu.VMEM((_OUT_DIM,), jnp.int32),
            pltpu.VMEM((_P + 1, _OUT_DIM), jnp.float32),
            pltpu.VMEM(((_P + 1) * _OUT_DIM,), jnp.int32),
            pltpu.VMEM((_GBUF,), jnp.float32),
            pltpu.VMEM((_GBUF,), jnp.float32),
            pltpu.SemaphoreType.DMA,
        ],
        compiler_params=pltpu.CompilerParams(needs_layout_passes=False),
    )
    return f(ctrl_t, uspan, nut)


def kernel(ctrl_pts, uspan, Nu):
    ctrl_t = jnp.transpose(ctrl_pts, (1, 2, 0)).reshape(_ROWS, _BATCH)
    nut = Nu.T
    out = _sc_eval(ctrl_t, uspan, nut)
    return jnp.transpose(out.reshape(_DIM, _BATCH, _OUT_DIM), (1, 2, 0))


# speedup vs baseline: 1.0801x; 1.0801x over previous
import functools

import jax
import jax.numpy as jnp
from jax import lax
from jax.experimental import pallas as pl
from jax.experimental.pallas import tpu as pltpu
from jax.experimental.pallas import tpu_sc as plsc

_BATCH = 1024
_M = 64
_P = 3
_DIM = 3
_OUT_DIM = 1024

_NC = 2
_NS = 16
_L = 16
_NW = _NC * _NS
_B_PER_W = _BATCH // _NW
_SLAB = 128

_N_CHUNK = _OUT_DIM // _L
_N_G = 4
_B_PER_G = _B_PER_W // _N_G

_ROWS = _M * (_DIM + 1)


def _sc_body(ctrl_hbm, uspan_hbm, nut_hbm, out_hbm, ctrl_v, ctrl_f, uspan_v,
             nut_v, idx_v, out_v0, out_v1, sem):
    wid = lax.axis_index("s") * _NC + lax.axis_index("c")
    base = wid * _B_PER_W
    bofs = (wid % (_SLAB // _B_PER_W)) * _B_PER_W

    pltpu.sync_copy(ctrl_hbm.at[:, pl.ds((base // _SLAB) * _SLAB, _SLAB)],
                    ctrl_v)
    pltpu.sync_copy(uspan_hbm, uspan_v)
    pltpu.sync_copy(nut_hbm, nut_v)

    @plsc.parallel_loop(0, _M)
    def _flatten(m):
        for d in range(_DIM + 1):
            for g in range(_B_PER_W // _L):
                ctrl_f[pl.ds(m * 128 + d * 32 + g * _L, _L)] = (
                    ctrl_v[m * (_DIM + 1) + d, pl.ds(bofs + g * _L, _L)])

    @plsc.parallel_loop(0, _N_CHUNK)
    def _pre(c):
        u = uspan_v[pl.ds(c * _L, _L)]
        for j in range(_P + 1):
            idx_v[pl.ds((c * (_P + 1) + j) * _L, _L)] = (u + (j - _P)) * 128

    def run_group(g, buf):
        @plsc.parallel_loop(0, _N_CHUNK)
        def c_body(c):
            mj = [idx_v[pl.ds((c * (_P + 1) + j) * _L, _L)]
                  for j in range(_P + 1)]
            nu = [nut_v[j, pl.ds(c * _L, _L)] for j in range(_P + 1)]
            cofs = c * _L

            @plsc.parallel_loop(0, _B_PER_G, unroll=2)
            def _bloop(b):
                bv = jnp.full((_L,), g * _B_PER_G + b, jnp.int32)
                mb = [mj[j] + bv for j in range(_P + 1)]
                w = plsc.load_gather(ctrl_f, [mb[0] + _DIM * 32])
                a = nu[0] * w
                denom = a
                num0 = a * plsc.load_gather(ctrl_f, [mb[0]])
                num1 = a * plsc.load_gather(ctrl_f, [mb[0] + 32])
                num2 = a * plsc.load_gather(ctrl_f, [mb[0] + 64])
                for j in range(1, _P + 1):
                    w = plsc.load_gather(ctrl_f, [mb[j] + _DIM * 32])
                    a = nu[j] * w
                    denom = denom + a
                    num0 = num0 + a * plsc.load_gather(ctrl_f, [mb[j]])
                    num1 = num1 + a * plsc.load_gather(ctrl_f, [mb[j] + 32])
                    num2 = num2 + a * plsc.load_gather(ctrl_f, [mb[j] + 64])
                inv = 1.0 / denom
                buf[0, b, pl.ds(cofs, _L)] = num0 * inv
                buf[1, b, pl.ds(cofs, _L)] = num1 * inv
                buf[2, b, pl.ds(cofs, _L)] = num2 * inv

        return [pltpu.async_copy(
            buf.at[d],
            out_hbm.at[d, pl.ds(base + g * _B_PER_G, _B_PER_G), :], sem)
            for d in range(_DIM)]

    bufs = (out_v0, out_v1)
    pending = []
    for g in range(_N_G):
        if g >= 2:
            for cp in pending[g - 2]:
                cp.wait()
        pending.append(run_group(g, bufs[g % 2]))
    for cps in pending[-2:]:
        for cp in cps:
            cp.wait()


@jax.jit
def _sc_eval(ctrl_t, uspan, nut):
    mesh = plsc.VectorSubcoreMesh(core_axis_name="c", subcore_axis_name="s",
                                  num_cores=_NC, num_subcores=_NS)
    f = pl.kernel(
        _sc_body,
        out_type=jax.ShapeDtypeStruct((_DIM, _BATCH, _OUT_DIM), jnp.float32),
        mesh=mesh,
        scratch_types=[
            pltpu.VMEM((_ROWS, _SLAB), jnp.float32),
            pltpu.VMEM((_ROWS * _B_PER_W,), jnp.float32),
            pltpu.VMEM((_OUT_DIM,), jnp.int32),
            pltpu.VMEM((_P + 1, _OUT_DIM), jnp.float32),
            pltpu.VMEM(((_P + 1) * _OUT_DIM,), jnp.int32),
            pltpu.VMEM((_DIM, _B_PER_G, _OUT_DIM), jnp.float32),
            pltpu.VMEM((_DIM, _B_PER_G, _OUT_DIM), jnp.float32),
            pltpu.SemaphoreType.DMA,
        ],
        compiler_params=pltpu.CompilerParams(needs_layout_passes=False),
    )
    return f(ctrl_t, uspan, nut)


def kernel(ctrl_pts, uspan, Nu):
    ctrl_t = jnp.transpose(ctrl_pts, (1, 2, 0)).reshape(_ROWS, _BATCH)
    nut = Nu.T
    out = _sc_eval(ctrl_t, uspan, nut)
    return jnp.transpose(out, (1, 2, 0))
